# baseline (speedup 1.0000x reference)
import functools

import jax
import jax.numpy as jnp
from jax import lax
from jax.experimental import pallas as pl
from jax.experimental.pallas import tpu as pltpu
from jax.experimental.pallas import tpu_sc as plsc

N = 10000
E = 320000
D = 128
H = 128
U = 128
OUT = 64

NC = 2
NS = 16
NW = NC * NS
EPW = E // NW
CH = 50
NCHUNK = EPW // CH
PH = 10
PC = NCHUNK // PH
NBUF = 5
DRT = 5
RPT = N // DRT

BN = 1000
NBLK = N // BN


def _sc_body(x_hbm, ei_hbm, sum_hbm, deg_hbm,
             srcA_v, srcB_v, dstA_v, dstB_v,
             rows0_v, rows1_v, rows2_v, rows3_v, rows4_v, ones_v, zdeg_v,
             sum_sh, deg_sh,
             semg0, semg1, semg2, semg3, semg4,
             sems0, sems1, sems2, sems3, sems4,
             semi, semd):
    c = lax.axis_index("c")
    s = lax.axis_index("s")
    wid = s * NC + c
    rows = (rows0_v, rows1_v, rows2_v, rows3_v, rows4_v)
    semg = (semg0, semg1, semg2, semg3, semg4)
    sems = (sems0, sems1, sems2, sems3, sems4)

    z16 = jnp.zeros((16,), jnp.float32)
    o16 = jnp.ones((16,), jnp.float32)
    for k in range(64 // 16):
        ones_v[pl.ds(16 * k, 16)] = o16
    for r in range(40):
        for k in range(D // 16):
            rows0_v[r, pl.ds(16 * k, 16)] = z16
    for k in range(1000 // 16):
        zdeg_v[pl.ds(16 * k, 16)] = z16
    ones = ones_v.at[pl.ds(0, CH)]

    @pl.when(s < DRT)
    def _():
        for k in range(RPT // 40):
            pltpu.sync_copy(rows0_v.at[pl.ds(0, 40)],
                            sum_sh.at[pl.ds(s * RPT + 40 * k, 40)])

    @pl.when(s == 0)
    def _():
        for k in range(N // 1000):
            pltpu.sync_copy(zdeg_v, deg_sh.at[pl.ds(1000 * k, 1000)])

    plsc.subcore_barrier()

    pltpu.sync_copy(ei_hbm.at[0, wid, 0], srcA_v)
    pltpu.sync_copy(ei_hbm.at[1, wid, 0], dstA_v)
    src_bufs = (srcA_v, srcB_v)
    dst_bufs = (dstA_v, dstB_v)
    for k in range(NBUF - 1):
        pltpu.async_copy(x_hbm.at[srcA_v.at[k]], rows[k], semg[k])

    for ph in range(PH):
        sa = src_bufs[ph % 2]
        da = dst_bufs[ph % 2]
        sb = src_bufs[(ph + 1) % 2]
        db = dst_bufs[(ph + 1) % 2]
        if ph + 1 < PH:
            pltpu.async_copy(ei_hbm.at[0, wid, ph + 1], sb, semi)
            pltpu.async_copy(ei_hbm.at[1, wid, ph + 1], db, semi)

        def quad(q, carry, sa=sa, da=da, ph=ph):
            for k in range(NBUF):
                j = NBUF * q + k
                b = k
                bp = (k + NBUF - 1) % NBUF
                pltpu.make_async_copy(x_hbm.at[sa.at[j]], rows[b],
                                      semg[b]).wait()
                pltpu.async_copy(rows[b], sum_sh.at[da.at[j]], sems[b],
                                 add=True)
                pltpu.async_copy(ones, deg_sh.at[da.at[j]], semd, add=True)

                def wait_prev(bp=bp, da=da):
                    pltpu.make_async_copy(rows[bp], sum_sh.at[da.at[0]],
                                          sems[bp]).wait()

                if k == 0:
                    if ph == 0:
                        pl.when(q > 0)(wait_prev)
                    else:
                        wait_prev()
                else:
                    wait_prev()

                @pl.when(j + NBUF - 1 <= PC - 1)
                def _(sa=sa, j=j, bp=bp):
                    pltpu.async_copy(x_hbm.at[sa.at[j + NBUF - 1]], rows[bp],
                                     semg[bp])

            return carry

        lax.fori_loop(0, PC // NBUF, quad, 0)

        if ph + 1 < PH:
            pltpu.make_async_copy(ei_hbm.at[0, wid, ph + 1], sb, semi).wait()
            pltpu.make_async_copy(ei_hbm.at[1, wid, ph + 1], db, semi).wait()
            for k in range(NBUF - 1):
                pltpu.async_copy(x_hbm.at[sb.at[k]], rows[k], semg[k])
        if ph == PH - 1:
            pltpu.make_async_copy(rows[NBUF - 1], sum_sh.at[da.at[0]],
                                  sems[NBUF - 1]).wait()

        def degdrain(_, carry, da=da):
            pltpu.make_async_copy(ones, deg_sh.at[da.at[0]], semd).wait()
            return carry

        lax.fori_loop(0, PC, degdrain, 0)

    plsc.subcore_barrier()

    @pl.when(s < DRT)
    def _():
        pltpu.sync_copy(sum_sh.at[pl.ds(s * RPT, RPT)],
                        sum_hbm.at[c, pl.ds(s * RPT, RPT)])

    @pl.when(s == 0)
    def _():
        pltpu.sync_copy(deg_sh, deg_hbm.at[c])


@jax.jit
def _segsum(x, ei5):
    mesh = plsc.VectorSubcoreMesh(core_axis_name="c", subcore_axis_name="s")
    k = pl.kernel(
        _sc_body,
        out_type=(jax.ShapeDtypeStruct((NC, N, D), jnp.float32),
                  jax.ShapeDtypeStruct((NC, N), jnp.float32)),
        mesh=mesh,
        scratch_types=[
            pltpu.VMEM((PC, CH), jnp.int32),
            pltpu.VMEM((PC, CH), jnp.int32),
            pltpu.VMEM((PC, CH), jnp.int32),
            pltpu.VMEM((PC, CH), jnp.int32),
            pltpu.VMEM((CH, D), jnp.float32),
            pltpu.VMEM((CH, D), jnp.float32),
            pltpu.VMEM((CH, D), jnp.float32),
            pltpu.VMEM((CH, D), jnp.float32),
            pltpu.VMEM((CH, D), jnp.float32),
            pltpu.VMEM((64,), jnp.float32),
            pltpu.VMEM((1000,), jnp.float32),
            pltpu.VMEM_SHARED((N, D), jnp.float32),
            pltpu.VMEM_SHARED((N,), jnp.float32),
            pltpu.SemaphoreType.DMA,
            pltpu.SemaphoreType.DMA,
            pltpu.SemaphoreType.DMA,
            pltpu.SemaphoreType.DMA,
            pltpu.SemaphoreType.DMA,
            pltpu.SemaphoreType.DMA,
            pltpu.SemaphoreType.DMA,
            pltpu.SemaphoreType.DMA,
            pltpu.SemaphoreType.DMA,
            pltpu.SemaphoreType.DMA,
            pltpu.SemaphoreType.DMA,
            pltpu.SemaphoreType.DMA,
        ],
    )
    return k(x, ei5)


def _tc_body(x_ref, sum_ref, deg_ref, ws_ref, wn_ref, b_ref,
             wu1_ref, bu1_ref, wu2_ref, bu2_ref, out_ref, acc_ref):
    i = pl.program_id(0)

    @pl.when(i == 0)
    def _():
        acc_ref[...] = jnp.zeros_like(acc_ref)

    S = sum_ref[0] + sum_ref[1]
    deg = deg_ref[0, 0, 0, :] + deg_ref[1, 0, 0, :]
    inv = 1.0 / jnp.maximum(deg, 1.0)
    Sn = S * inv[:, None]
    h = x_ref[...] @ ws_ref[...] + Sn @ wn_ref[...] + b_ref[...]
    h = jnp.maximum(h, 0.0)
    acc_ref[...] += jnp.sum(h, axis=0, keepdims=True)

    @pl.when(i == NBLK - 1)
    def _():
        u = acc_ref[...] * (1.0 / N)
        u = jnp.maximum(u @ wu1_ref[...] + bu1_ref[...], 0.0)
        out_ref[...] = u @ wu2_ref[...] + bu2_ref[...]


@jax.jit
def _dense(x, sumP, degP4, W_self, W_nbr, b2, W_u1, b1u, W_u2, b2u):
    return pl.pallas_call(
        _tc_body,
        grid=(NBLK,),
        in_specs=[
            pl.BlockSpec((BN, D), lambda i: (i, 0)),
            pl.BlockSpec((NC, BN, D), lambda i: (0, i, 0)),
            pl.BlockSpec((NC, 1, 1, BN), lambda i: (0, i, 0, 0)),
            pl.BlockSpec((D, H), lambda i: (0, 0)),
            pl.BlockSpec((D, H), lambda i: (0, 0)),
            pl.BlockSpec((1, H), lambda i: (0, 0)),
            pl.BlockSpec((H, U), lambda i: (0, 0)),
            pl.BlockSpec((1, U), lambda i: (0, 0)),
            pl.BlockSpec((U, OUT), lambda i: (0, 0)),
            pl.BlockSpec((1, OUT), lambda i: (0, 0)),
        ],
        out_specs=pl.BlockSpec((1, OUT), lambda i: (0, 0)),
        out_shape=jax.ShapeDtypeStruct((1, OUT), jnp.float32),
        scratch_shapes=[pltpu.VMEM((1, H), jnp.float32)],
    )(x, sumP, degP4, W_self, W_nbr, b2, W_u1, b1u, W_u2, b2u)


def kernel(x, edge_index, W_self, W_nbr, b_extr, W_u1, b_u1, W_u2, b_u2):
    ei5 = edge_index.astype(jnp.int32).reshape(2, NW, PH, PC, CH)
    sumP, degP = _segsum(x, ei5)
    degP4 = degP.reshape(NC, NBLK, 1, BN)
    val = _dense(x, sumP, degP4, W_self, W_nbr,
                 b_extr.reshape(1, H), W_u1, b_u1.reshape(1, U),
                 W_u2, b_u2.reshape(1, OUT))
    return val.reshape(OUT)


# speedup vs baseline: 1.0250x; 1.0250x over previous
import functools

import jax
import jax.numpy as jnp
from jax import lax
from jax.experimental import pallas as pl
from jax.experimental.pallas import tpu as pltpu
from jax.experimental.pallas import tpu_sc as plsc

N = 10000
E = 320000
D = 128
H = 128
U = 128
OUT = 64

NC = 2
NS = 16
NW = NC * NS
EPW = E // NW
CH = 40
NCHUNK = EPW // CH
PH = 5
PC = NCHUNK // PH
NBUF = 5
DRT = 5
RPT = N // DRT

BN = 1000
NBLK = N // BN


def _sc_body(x_hbm, src_hbm, dst_hbm, sum_hbm, deg_hbm,
             srcA_v, srcB_v, dstA_v, dstB_v,
             rows0_v, rows1_v, rows2_v, rows3_v, rows4_v, ones_v, zdeg_v,
             sum_sh, deg_sh,
             semg0, semg1, semg2, semg3, semg4,
             sems0, sems1, sems2, sems3, sems4,
             semi, semd):
    c = lax.axis_index("c")
    s = lax.axis_index("s")
    wid = s * NC + c
    rows = (rows0_v, rows1_v, rows2_v, rows3_v, rows4_v)
    semg = (semg0, semg1, semg2, semg3, semg4)
    sems = (sems0, sems1, sems2, sems3, sems4)

    z16 = jnp.zeros((16,), jnp.float32)
    o16 = jnp.ones((16,), jnp.float32)
    for k in range(64 // 16):
        ones_v[pl.ds(16 * k, 16)] = o16
    for r in range(40):
        for k in range(D // 16):
            rows0_v[r, pl.ds(16 * k, 16)] = z16
    for k in range(1000 // 16):
        zdeg_v[pl.ds(16 * k, 16)] = z16
    ones = ones_v.at[pl.ds(0, CH)]

    @pl.when(s < DRT)
    def _():
        for k in range(RPT // 40):
            pltpu.sync_copy(rows0_v.at[pl.ds(0, 40)],
                            sum_sh.at[pl.ds(s * RPT + 40 * k, 40)])

    @pl.when(s == 0)
    def _():
        for k in range(N // 1000):
            pltpu.sync_copy(zdeg_v, deg_sh.at[pl.ds(1000 * k, 1000)])

    plsc.subcore_barrier()

    pltpu.sync_copy(src_hbm.at[pl.ds(wid * EPW, PC * CH)], srcA_v)
    pltpu.sync_copy(dst_hbm.at[pl.ds(wid * EPW, PC * CH)], dstA_v)
    src_bufs = (srcA_v, srcB_v)
    dst_bufs = (dstA_v, dstB_v)
    for k in range(NBUF - 1):
        pltpu.async_copy(x_hbm.at[srcA_v.at[pl.ds(k * CH, CH)]],
                         rows[k], semg[k])

    for ph in range(PH):
        sa = src_bufs[ph % 2]
        da = dst_bufs[ph % 2]
        sb = src_bufs[(ph + 1) % 2]
        db = dst_bufs[(ph + 1) % 2]
        nxt = pl.ds(wid * EPW + (ph + 1) * PC * CH, PC * CH)
        if ph + 1 < PH:
            pltpu.async_copy(src_hbm.at[nxt], sb, semi)
            pltpu.async_copy(dst_hbm.at[nxt], db, semi)

        def quad(q, carry, sa=sa, da=da, ph=ph):
            for k in range(NBUF):
                j = NBUF * q + k
                b = k
                bp = (k + NBUF - 1) % NBUF
                pltpu.make_async_copy(x_hbm.at[sa.at[pl.ds(j * CH, CH)]], rows[b],
                                      semg[b]).wait()
                pltpu.async_copy(rows[b], sum_sh.at[da.at[pl.ds(j * CH, CH)]], sems[b],
                                 add=True)
                pltpu.async_copy(ones, deg_sh.at[da.at[pl.ds(j * CH, CH)]], semd, add=True)

                def wait_prev(bp=bp, da=da):
                    pltpu.make_async_copy(rows[bp],
                                          sum_sh.at[da.at[pl.ds(0, CH)]],
                                          sems[bp]).wait()

                if k == 0:
                    if ph == 0:
                        pl.when(q > 0)(wait_prev)
                    else:
                        wait_prev()
                else:
                    wait_prev()

                @pl.when(j + NBUF - 1 <= PC - 1)
                def _(sa=sa, j=j, bp=bp):
                    pltpu.async_copy(
                        x_hbm.at[sa.at[pl.ds((j + NBUF - 1) * CH, CH)]],
                        rows[bp], semg[bp])

            return carry

        lax.fori_loop(0, PC // NBUF, quad, 0)

        if ph + 1 < PH:
            pltpu.make_async_copy(src_hbm.at[nxt], sb, semi).wait()
            pltpu.make_async_copy(dst_hbm.at[nxt], db, semi).wait()
            for k in range(NBUF - 1):
                pltpu.async_copy(x_hbm.at[sb.at[pl.ds(k * CH, CH)]],
                                 rows[k], semg[k])
        if ph == PH - 1:
            pltpu.make_async_copy(rows[NBUF - 1],
                                  sum_sh.at[da.at[pl.ds(0, CH)]],
                                  sems[NBUF - 1]).wait()

        def degdrain(_, carry, da=da):
            pltpu.make_async_copy(ones, deg_sh.at[da.at[pl.ds(0, CH)]],
                                  semd).wait()
            return carry

        lax.fori_loop(0, PC, degdrain, 0)

    plsc.subcore_barrier()

    @pl.when(s < DRT)
    def _():
        pltpu.sync_copy(sum_sh.at[pl.ds(s * RPT, RPT)],
                        sum_hbm.at[c, pl.ds(s * RPT, RPT)])

    @pl.when(s == 0)
    def _():
        pltpu.sync_copy(deg_sh, deg_hbm.at[c])


@jax.jit
def _segsum(x, src1, dst1):
    mesh = plsc.VectorSubcoreMesh(core_axis_name="c", subcore_axis_name="s")
    k = pl.kernel(
        _sc_body,
        out_type=(jax.ShapeDtypeStruct((NC, N, D), jnp.float32),
                  jax.ShapeDtypeStruct((NC, N), jnp.float32)),
        mesh=mesh,
        scratch_types=[
            pltpu.VMEM((PC * CH,), jnp.int32),
            pltpu.VMEM((PC * CH,), jnp.int32),
            pltpu.VMEM((PC * CH,), jnp.int32),
            pltpu.VMEM((PC * CH,), jnp.int32),
            pltpu.VMEM((CH, D), jnp.float32),
            pltpu.VMEM((CH, D), jnp.float32),
            pltpu.VMEM((CH, D), jnp.float32),
            pltpu.VMEM((CH, D), jnp.float32),
            pltpu.VMEM((CH, D), jnp.float32),
            pltpu.VMEM((64,), jnp.float32),
            pltpu.VMEM((1000,), jnp.float32),
            pltpu.VMEM_SHARED((N, D), jnp.float32),
            pltpu.VMEM_SHARED((N,), jnp.float32),
            pltpu.SemaphoreType.DMA,
            pltpu.SemaphoreType.DMA,
            pltpu.SemaphoreType.DMA,
            pltpu.SemaphoreType.DMA,
            pltpu.SemaphoreType.DMA,
            pltpu.SemaphoreType.DMA,
            pltpu.SemaphoreType.DMA,
            pltpu.SemaphoreType.DMA,
            pltpu.SemaphoreType.DMA,
            pltpu.SemaphoreType.DMA,
            pltpu.SemaphoreType.DMA,
            pltpu.SemaphoreType.DMA,
        ],
    )
    return k(x, src1, dst1)


def _tc_body(x_ref, sum_ref, deg_ref, ws_ref, wn_ref, b_ref,
             wu1_ref, bu1_ref, wu2_ref, bu2_ref, out_ref, acc_ref):
    i = pl.program_id(0)

    @pl.when(i == 0)
    def _():
        acc_ref[...] = jnp.zeros_like(acc_ref)

    S = sum_ref[0] + sum_ref[1]
    deg = deg_ref[0, 0, 0, :] + deg_ref[1, 0, 0, :]
    inv = 1.0 / jnp.maximum(deg, 1.0)
    Sn = S * inv[:, None]
    h = x_ref[...] @ ws_ref[...] + Sn @ wn_ref[...] + b_ref[...]
    h = jnp.maximum(h, 0.0)
    acc_ref[...] += jnp.sum(h, axis=0, keepdims=True)

    @pl.when(i == NBLK - 1)
    def _():
        u = acc_ref[...] * (1.0 / N)
        u = jnp.maximum(u @ wu1_ref[...] + bu1_ref[...], 0.0)
        out_ref[...] = u @ wu2_ref[...] + bu2_ref[...]


@jax.jit
def _dense(x, sumP, degP4, W_self, W_nbr, b2, W_u1, b1u, W_u2, b2u):
    return pl.pallas_call(
        _tc_body,
        grid=(NBLK,),
        in_specs=[
            pl.BlockSpec((BN, D), lambda i: (i, 0)),
            pl.BlockSpec((NC, BN, D), lambda i: (0, i, 0)),
            pl.BlockSpec((NC, 1, 1, BN), lambda i: (0, i, 0, 0)),
            pl.BlockSpec((D, H), lambda i: (0, 0)),
            pl.BlockSpec((D, H), lambda i: (0, 0)),
            pl.BlockSpec((1, H), lambda i: (0, 0)),
            pl.BlockSpec((H, U), lambda i: (0, 0)),
            pl.BlockSpec((1, U), lambda i: (0, 0)),
            pl.BlockSpec((U, OUT), lambda i: (0, 0)),
            pl.BlockSpec((1, OUT), lambda i: (0, 0)),
        ],
        out_specs=pl.BlockSpec((1, OUT), lambda i: (0, 0)),
        out_shape=jax.ShapeDtypeStruct((1, OUT), jnp.float32),
        scratch_shapes=[pltpu.VMEM((1, H), jnp.float32)],
    )(x, sumP, degP4, W_self, W_nbr, b2, W_u1, b1u, W_u2, b2u)


def kernel(x, edge_index, W_self, W_nbr, b_extr, W_u1, b_u1, W_u2, b_u2):
    ei = edge_index.astype(jnp.int32)
    sumP, degP = _segsum(x, ei[0], ei[1])
    degP4 = degP.reshape(NC, NBLK, 1, BN)
    val = _dense(x, sumP, degP4, W_self, W_nbr,
                 b_extr.reshape(1, H), W_u1, b_u1.reshape(1, U),
                 W_u2, b_u2.reshape(1, OUT))
    return val.reshape(OUT)


# speedup vs baseline: 1.0425x; 1.0171x over previous
import functools

import jax
import jax.numpy as jnp
from jax import lax
from jax.experimental import pallas as pl
from jax.experimental.pallas import tpu as pltpu
from jax.experimental.pallas import tpu_sc as plsc

N = 10000
E = 320000
D = 128
H = 128
U = 128
OUT = 64

NC = 2
NS = 16
NW = NC * NS
EPW = E // NW
CH = 40
NCHUNK = EPW // CH
PH = 5
PC = NCHUNK // PH
NBUF = 5
DRT = 5
RPT = N // DRT

BN = 2000
NBLK = N // BN


def _sc_body(x_hbm, src_hbm, dst_hbm, sum_hbm, deg_hbm,
             srcA_v, srcB_v, dstA_v, dstB_v,
             rows0_v, rows1_v, rows2_v, rows3_v, rows4_v, ones_v, zdeg_v,
             sum_sh, deg_sh,
             semg0, semg1, semg2, semg3, semg4,
             sems0, sems1, sems2, sems3, sems4,
             semi, semd):
    c = lax.axis_index("c")
    s = lax.axis_index("s")
    wid = s * NC + c
    rows = (rows0_v, rows1_v, rows2_v, rows3_v, rows4_v)
    semg = (semg0, semg1, semg2, semg3, semg4)
    sems = (sems0, sems1, sems2, sems3, sems4)

    z16 = jnp.zeros((16,), jnp.float32)
    o16 = jnp.ones((16,), jnp.float32)
    for k in range(64 // 16):
        ones_v[pl.ds(16 * k, 16)] = o16
    for r in range(40):
        for k in range(D // 16):
            rows0_v[r, pl.ds(16 * k, 16)] = z16
    for k in range(1000 // 16):
        zdeg_v[pl.ds(16 * k, 16)] = z16
    ones = ones_v.at[pl.ds(0, CH)]

    @pl.when(s < DRT)
    def _():
        for k in range(RPT // 40):
            pltpu.sync_copy(rows0_v.at[pl.ds(0, 40)],
                            sum_sh.at[pl.ds(s * RPT + 40 * k, 40)])

    @pl.when(s == 0)
    def _():
        for k in range(N // 1000):
            pltpu.sync_copy(zdeg_v, deg_sh.at[pl.ds(1000 * k, 1000)])

    plsc.subcore_barrier()

    pltpu.sync_copy(src_hbm.at[pl.ds(wid * EPW, PC * CH)], srcA_v)
    pltpu.sync_copy(dst_hbm.at[pl.ds(wid * EPW, PC * CH)], dstA_v)
    src_bufs = (srcA_v, srcB_v)
    dst_bufs = (dstA_v, dstB_v)
    for k in range(NBUF - 1):
        pltpu.async_copy(x_hbm.at[srcA_v.at[pl.ds(k * CH, CH)]],
                         rows[k], semg[k])

    for ph in range(PH):
        sa = src_bufs[ph % 2]
        da = dst_bufs[ph % 2]
        sb = src_bufs[(ph + 1) % 2]
        db = dst_bufs[(ph + 1) % 2]
        nxt = pl.ds(wid * EPW + (ph + 1) * PC * CH, PC * CH)
        if ph + 1 < PH:
            pltpu.async_copy(src_hbm.at[nxt], sb, semi)
            pltpu.async_copy(dst_hbm.at[nxt], db, semi)

        def quad(q, carry, sa=sa, da=da, ph=ph):
            for k in range(NBUF):
                j = NBUF * q + k
                b = k
                bp = (k + NBUF - 1) % NBUF
                pltpu.make_async_copy(x_hbm.at[sa.at[pl.ds(j * CH, CH)]], rows[b],
                                      semg[b]).wait()
                pltpu.async_copy(rows[b], sum_sh.at[da.at[pl.ds(j * CH, CH)]], sems[b],
                                 add=True)
                pltpu.async_copy(ones, deg_sh.at[da.at[pl.ds(j * CH, CH)]], semd, add=True)

                def wait_prev(bp=bp, da=da):
                    pltpu.make_async_copy(rows[bp],
                                          sum_sh.at[da.at[pl.ds(0, CH)]],
                                          sems[bp]).wait()

                if k == 0:
                    if ph == 0:
                        pl.when(q > 0)(wait_prev)
                    else:
                        wait_prev()
                else:
                    wait_prev()

                @pl.when(j + NBUF - 1 <= PC - 1)
                def _(sa=sa, j=j, bp=bp):
                    pltpu.async_copy(
                        x_hbm.at[sa.at[pl.ds((j + NBUF - 1) * CH, CH)]],
                        rows[bp], semg[bp])

            return carry

        lax.fori_loop(0, PC // NBUF, quad, 0)

        if ph + 1 < PH:
            pltpu.make_async_copy(src_hbm.at[nxt], sb, semi).wait()
            pltpu.make_async_copy(dst_hbm.at[nxt], db, semi).wait()
            for k in range(NBUF - 1):
                pltpu.async_copy(x_hbm.at[sb.at[pl.ds(k * CH, CH)]],
                                 rows[k], semg[k])
        if ph == PH - 1:
            pltpu.make_async_copy(rows[NBUF - 1],
                                  sum_sh.at[da.at[pl.ds(0, CH)]],
                                  sems[NBUF - 1]).wait()

        def degdrain(_, carry, da=da):
            pltpu.make_async_copy(ones, deg_sh.at[da.at[pl.ds(0, CH)]],
                                  semd).wait()
            return carry

        lax.fori_loop(0, PC, degdrain, 0)

    plsc.subcore_barrier()

    @pl.when(s < DRT)
    def _():
        pltpu.sync_copy(sum_sh.at[pl.ds(s * RPT, RPT)],
                        sum_hbm.at[c, pl.ds(s * RPT, RPT)])

    @pl.when(s == 0)
    def _():
        pltpu.sync_copy(deg_sh, deg_hbm.at[c])


@jax.jit
def _segsum(x, src1, dst1):
    mesh = plsc.VectorSubcoreMesh(core_axis_name="c", subcore_axis_name="s")
    k = pl.kernel(
        _sc_body,
        out_type=(jax.ShapeDtypeStruct((NC, N, D), jnp.float32),
                  jax.ShapeDtypeStruct((NC, N), jnp.float32)),
        mesh=mesh,
        scratch_types=[
            pltpu.VMEM((PC * CH,), jnp.int32),
            pltpu.VMEM((PC * CH,), jnp.int32),
            pltpu.VMEM((PC * CH,), jnp.int32),
            pltpu.VMEM((PC * CH,), jnp.int32),
            pltpu.VMEM((CH, D), jnp.float32),
            pltpu.VMEM((CH, D), jnp.float32),
            pltpu.VMEM((CH, D), jnp.float32),
            pltpu.VMEM((CH, D), jnp.float32),
            pltpu.VMEM((CH, D), jnp.float32),
            pltpu.VMEM((64,), jnp.float32),
            pltpu.VMEM((1000,), jnp.float32),
            pltpu.VMEM_SHARED((N, D), jnp.float32),
            pltpu.VMEM_SHARED((N,), jnp.float32),
            pltpu.SemaphoreType.DMA,
            pltpu.SemaphoreType.DMA,
            pltpu.SemaphoreType.DMA,
            pltpu.SemaphoreType.DMA,
            pltpu.SemaphoreType.DMA,
            pltpu.SemaphoreType.DMA,
            pltpu.SemaphoreType.DMA,
            pltpu.SemaphoreType.DMA,
            pltpu.SemaphoreType.DMA,
            pltpu.SemaphoreType.DMA,
            pltpu.SemaphoreType.DMA,
            pltpu.SemaphoreType.DMA,
        ],
    )
    return k(x, src1, dst1)


def _tc_body(x_ref, sum_ref, deg_ref, ws_ref, wn_ref, b_ref,
             wu1_ref, bu1_ref, wu2_ref, bu2_ref, out_ref, acc_ref):
    i = pl.program_id(0)

    @pl.when(i == 0)
    def _():
        acc_ref[...] = jnp.zeros_like(acc_ref)

    S = sum_ref[0] + sum_ref[1]
    deg = deg_ref[0, 0, 0, :] + deg_ref[1, 0, 0, :]
    inv = 1.0 / jnp.maximum(deg, 1.0)
    Sn = S * inv[:, None]
    h = x_ref[...] @ ws_ref[...] + Sn @ wn_ref[...] + b_ref[...]
    h = jnp.maximum(h, 0.0)
    acc_ref[...] += jnp.sum(h, axis=0, keepdims=True)

    @pl.when(i == NBLK - 1)
    def _():
        u = acc_ref[...] * (1.0 / N)
        u = jnp.maximum(u @ wu1_ref[...] + bu1_ref[...], 0.0)
        out_ref[...] = u @ wu2_ref[...] + bu2_ref[...]


@jax.jit
def _dense(x, sumP, degP4, W_self, W_nbr, b2, W_u1, b1u, W_u2, b2u):
    return pl.pallas_call(
        _tc_body,
        grid=(NBLK,),
        in_specs=[
            pl.BlockSpec((BN, D), lambda i: (i, 0)),
            pl.BlockSpec((NC, BN, D), lambda i: (0, i, 0)),
            pl.BlockSpec((NC, 1, 1, BN), lambda i: (0, i, 0, 0)),
            pl.BlockSpec((D, H), lambda i: (0, 0)),
            pl.BlockSpec((D, H), lambda i: (0, 0)),
            pl.BlockSpec((1, H), lambda i: (0, 0)),
            pl.BlockSpec((H, U), lambda i: (0, 0)),
            pl.BlockSpec((1, U), lambda i: (0, 0)),
            pl.BlockSpec((U, OUT), lambda i: (0, 0)),
            pl.BlockSpec((1, OUT), lambda i: (0, 0)),
        ],
        out_specs=pl.BlockSpec((1, OUT), lambda i: (0, 0)),
        out_shape=jax.ShapeDtypeStruct((1, OUT), jnp.float32),
        scratch_shapes=[pltpu.VMEM((1, H), jnp.float32)],
    )(x, sumP, degP4, W_self, W_nbr, b2, W_u1, b1u, W_u2, b2u)


def kernel(x, edge_index, W_self, W_nbr, b_extr, W_u1, b_u1, W_u2, b_u2):
    ei = edge_index.astype(jnp.int32)
    sumP, degP = _segsum(x, ei[0], ei[1])
    degP4 = degP.reshape(NC, NBLK, 1, BN)
    val = _dense(x, sumP, degP4, W_self, W_nbr,
                 b_extr.reshape(1, H), W_u1, b_u1.reshape(1, U),
                 W_u2, b_u2.reshape(1, OUT))
    return val.reshape(OUT)
